# baseline (speedup 1.0000x reference)
import jax
import jax.numpy as jnp
from jax import lax
from jax.experimental import pallas as pl
from jax.experimental.pallas import tpu as pltpu

_N, _C = 32, 768
_K = _C // 2
_ONE_BITS = 0x3F800000


def _mask_body(mask_ref, W1_ref, b1_ref, W2_ref, b2_ref, m_ref, cm_ref):
    h = jnp.dot(mask_ref[:], W1_ref[:], preferred_element_type=jnp.float32)
    h = jnp.maximum(h + b1_ref[:], 0.0)
    z = jnp.dot(h, W2_ref[:], preferred_element_type=jnp.float32) + b2_ref[:]
    m = jax.nn.sigmoid(z)
    m_ref[:] = m

    bits = lax.bitcast_convert_type(m, jnp.int32)

    def step(_, carry):
        lo, hi = carry
        mid = (lo + hi) >> 1
        cnt = jnp.sum((bits <= mid).astype(jnp.int32), axis=1, keepdims=True)
        ge = cnt >= _K
        return jnp.where(ge, lo, mid), jnp.where(ge, mid, hi)

    lo0 = jnp.full((_N, 1), -1, jnp.int32)
    hi0 = jnp.full((_N, 1), _ONE_BITS, jnp.int32)
    _, kth_bits = lax.fori_loop(0, 31, step, (lo0, hi0))
    cm_ref[:] = (bits > kth_bits).astype(jnp.float32)


def _swap_body(cm_ref, lst_ref, gui_ref, ol_ref, og_ref):
    cm = cm_ref[:] > 0.5
    l = lst_ref[:]
    g = gui_ref[:]
    ol_ref[:] = jnp.where(cm, g, l)
    og_ref[:] = jnp.where(cm, l, g)


def kernel(lst, gui, mask, W1, b1, W2, b2):
    N, C, H, W = lst.shape
    HW = H * W

    m, cm = pl.pallas_call(
        _mask_body,
        out_shape=(
            jax.ShapeDtypeStruct((N, C), jnp.float32),
            jax.ShapeDtypeStruct((N, C), jnp.float32),
        ),
    )(mask, W1, b1.reshape(1, C), W2, b2.reshape(1, C))

    R = 512
    lst2 = lst.reshape(N * C, HW)
    gui2 = gui.reshape(N * C, HW)
    cm2 = cm.reshape(N * C, 1)

    ol, og = pl.pallas_call(
        _swap_body,
        grid=(N * C // R,),
        in_specs=[
            pl.BlockSpec((R, 1), lambda i: (i, 0)),
            pl.BlockSpec((R, HW), lambda i: (i, 0)),
            pl.BlockSpec((R, HW), lambda i: (i, 0)),
        ],
        out_specs=(
            pl.BlockSpec((R, HW), lambda i: (i, 0)),
            pl.BlockSpec((R, HW), lambda i: (i, 0)),
        ),
        out_shape=(
            jax.ShapeDtypeStruct((N * C, HW), jnp.float32),
            jax.ShapeDtypeStruct((N * C, HW), jnp.float32),
        ),
        compiler_params=pltpu.CompilerParams(
            dimension_semantics=("arbitrary",),
        ),
    )(cm2, lst2, gui2)

    return ol.reshape(N, C, H, W), og.reshape(N, C, H, W), m
